# Initial kernel scaffold; baseline (speedup 1.0000x reference)
import jax
import jax.numpy as jnp
from jax.experimental import pallas as pl


def kernel(mask_logits, gaussian_label, occ_score, surface_distance, inside_flag, depth_residual, mask_2d, image_coords):
    raise NotImplementedError("write your pallas kernel here")


# speedup vs baseline: 14.1313x; 14.1313x over previous
import functools

import jax
import jax.numpy as jnp
from jax import lax
from jax.experimental import pallas as pl
from jax.experimental.pallas import tpu as pltpu
from jax.experimental.pallas import tpu_sc as plsc

B, N, H, W = 4, 100000, 512, 512
HW = H * W

NC, NS, LANES = 2, 16, 16
CHUNK = 128
CPW = 6272
NCH = CPW // CHUNK
NP = NS * CPW
IPC = B // NC
SLICE = IPC * HW // NS
ZCHUNK = 2048


@functools.partial(
    pl.kernel,
    mesh=plsc.VectorSubcoreMesh(core_axis_name="c", subcore_axis_name="s"),
    out_type=jax.ShapeDtypeStruct((B * HW,), jnp.float32),
    scratch_types=[
        pltpu.VMEM((CPW,), jnp.float32),
        pltpu.VMEM((CPW,), jnp.int32),
        pltpu.VMEM((CPW,), jnp.int32),
        pltpu.VMEM((IPC, NCH, CHUNK), jnp.float32),
        pltpu.VMEM((IPC, NCH, CHUNK), jnp.int32),
        pltpu.VMEM((ZCHUNK,), jnp.float32),
        pltpu.VMEM_SHARED((IPC * HW,), jnp.float32),
    ],
)
def _sc_heatmap(logits_hbm, y_hbm, x_hbm, heat_hbm,
                lbuf, ybuf, xbuf, vals, idx, zbuf, heat_sh):
    c = lax.axis_index("c")
    s = lax.axis_index("s")

    @pl.loop(0, ZCHUNK, step=LANES)
    def _zero(i):
        zbuf[pl.ds(i, LANES)] = jnp.zeros((LANES,), jnp.float32)

    for t in range(SLICE // ZCHUNK):
        pltpu.sync_copy(zbuf, heat_sh.at[pl.ds(s * SLICE + t * ZCHUNK, ZCHUNK)])
    plsc.subcore_barrier()

    for k in range(IPC):
        item = c * IPC + k
        base = item * NP + s * CPW
        pltpu.sync_copy(logits_hbm.at[pl.ds(base, CPW)], lbuf)
        pltpu.sync_copy(y_hbm.at[pl.ds(base, CPW)], ybuf)
        pltpu.sync_copy(x_hbm.at[pl.ds(base, CPW)], xbuf)

        def _compute(j, k=k):
            for t in range(0, CHUNK, LANES):
                off = j * CHUNK + t
                yv = ybuf[pl.ds(off, LANES)]
                xv = xbuf[pl.ds(off, LANES)]
                yv = jnp.minimum(jnp.maximum(yv, 0), H - 1)
                xv = jnp.minimum(jnp.maximum(xv, 0), W - 1)
                idx[k, j, pl.ds(t, LANES)] = yv * W + xv + k * HW
                lv = lbuf[pl.ds(off, LANES)]
                vals[k, j, pl.ds(t, LANES)] = 1.0 / (1.0 + jnp.exp(-lv))

        pl.loop(0, NCH)(_compute)

        def _scatter(j, k=k):
            pltpu.sync_copy(vals.at[k, j], heat_sh.at[idx.at[k, j]], add=True)

        pl.loop(0, NCH)(_scatter)

    plsc.subcore_barrier()
    out_base = c * (IPC * HW) + s * SLICE
    pltpu.sync_copy(heat_sh.at[pl.ds(s * SLICE, SLICE)],
                    heat_hbm.at[pl.ds(out_base, SLICE)])


def _tc_nspace_body(lg_ref, lb_ref, oc_ref, sd_ref, in_ref, dr_ref, out_ref):
    x = lg_ref[...]
    t = lb_ref[...]
    p = 1.0 / (1.0 + jnp.exp(-x))
    bce = jnp.maximum(x, 0.0) - x * t + jnp.log(1.0 + jnp.exp(-jnp.abs(x)))
    s_bce = jnp.sum(bce, axis=1, keepdims=True)
    inter = jnp.sum(p * t, axis=1, keepdims=True)
    s_p = jnp.sum(p, axis=1, keepdims=True)
    s_t = jnp.sum(t, axis=1, keepdims=True)
    geo = p * ((1.0 - oc_ref[...]) + jnp.maximum(sd_ref[...], 0.0)
               + (1.0 - in_ref[...]) + jnp.abs(dr_ref[...]))
    s_geo = jnp.sum(geo, axis=1, keepdims=True)
    cols = lax.broadcasted_iota(jnp.int32, (B, 128), 1)
    out_ref[...] = (jnp.where(cols == 0, s_bce, 0.0)
                    + jnp.where(cols == 1, inter, 0.0)
                    + jnp.where(cols == 2, s_p, 0.0)
                    + jnp.where(cols == 3, s_t, 0.0)
                    + jnp.where(cols == 4, s_geo, 0.0))


_tc_nspace = pl.pallas_call(
    _tc_nspace_body,
    out_shape=jax.ShapeDtypeStruct((B, 128), jnp.float32),
)


def _tc_max_body(h_ref, out_ref):
    m = jnp.max(h_ref[...], axis=1, keepdims=True)
    out_ref[...] = jnp.broadcast_to(jnp.maximum(m, 1.0), (B, 128))


_tc_max = pl.pallas_call(
    _tc_max_body,
    out_shape=jax.ShapeDtypeStruct((B, 128), jnp.float32),
)


_BH = 32768
_NBLK = HW // _BH


def _tc_image_body(h_ref, t_ref, m_ref, out_ref):
    j = pl.program_id(0)

    @pl.when(j == 0)
    def _():
        out_ref[...] = jnp.zeros((B, 128), jnp.float32)

    h = h_ref[...]
    t = t_ref[...]
    m = m_ref[:, :1]
    p = jnp.clip(h / m, 1e-12, 1.0 - 1e-12)
    lp = jnp.maximum(jnp.log(p), -100.0)
    ln = jnp.maximum(jnp.log(1.0 - p), -100.0)
    bce = -(t * lp + (1.0 - t) * ln)
    s_bce = jnp.sum(bce, axis=1, keepdims=True)
    s_ht = jnp.sum(h * t, axis=1, keepdims=True)
    s_h = jnp.sum(h, axis=1, keepdims=True)
    s_t = jnp.sum(t, axis=1, keepdims=True)
    cols = lax.broadcasted_iota(jnp.int32, (B, 128), 1)
    out_ref[...] += (jnp.where(cols == 0, s_bce, 0.0)
                     + jnp.where(cols == 1, s_ht, 0.0)
                     + jnp.where(cols == 2, s_h, 0.0)
                     + jnp.where(cols == 3, s_t, 0.0))


_tc_image = pl.pallas_call(
    _tc_image_body,
    grid=(_NBLK,),
    in_specs=[
        pl.BlockSpec((B, _BH), lambda j: (0, j)),
        pl.BlockSpec((B, _BH), lambda j: (0, j)),
        pl.BlockSpec((B, 128), lambda j: (0, 0)),
    ],
    out_specs=pl.BlockSpec((B, 128), lambda j: (0, 0)),
    out_shape=jax.ShapeDtypeStruct((B, 128), jnp.float32),
)


def kernel(mask_logits, gaussian_label, occ_score, surface_distance,
           inside_flag, depth_residual, mask_2d, image_coords):
    smooth = 1e-6
    lam_mask, lam_geo, lam_reproj, lam_sparse = 1.0, 0.2, 0.5, 0.05

    pad = NP - N
    logits_p = jnp.pad(mask_logits, ((0, 0), (0, pad)),
                       constant_values=-1e9).reshape(-1)
    coords = image_coords.astype(jnp.int32)
    y_p = jnp.pad(coords[:, :, 0], ((0, 0), (0, pad))).reshape(-1)
    x_p = jnp.pad(coords[:, :, 1], ((0, 0), (0, pad))).reshape(-1)

    heat = _sc_heatmap(logits_p, y_p, x_p).reshape(B, HW)

    o1 = _tc_nspace(mask_logits, gaussian_label, occ_score,
                    surface_distance, inside_flag, depth_residual)

    mask2d = mask_2d.reshape(B, HW)
    mx = _tc_max(heat)
    o2 = _tc_image(heat, mask2d, mx)

    s_bce, inter, s_p, s_t, s_geo = (o1[:, 0], o1[:, 1], o1[:, 2],
                                     o1[:, 3], o1[:, 4])
    bce_n = s_bce / N
    dice_n = 1.0 - (2.0 * inter + smooth) / (s_p + s_t + smooth)
    mask_loss = jnp.mean(bce_n + dice_n)
    geometry_loss = jnp.mean(s_geo / N)
    sparse_loss = jnp.mean(s_p / N)

    m = mx[:, 0]
    bce_i = o2[:, 0] / HW
    inter_i = o2[:, 1] / m
    sp_i = o2[:, 2] / m
    st_i = o2[:, 3]
    dice_i = 1.0 - (2.0 * inter_i + smooth) / (sp_i + st_i + smooth)
    reprojection_loss = jnp.mean(bce_i + dice_i)

    total = (lam_mask * mask_loss + lam_geo * geometry_loss
             + lam_reproj * reprojection_loss + lam_sparse * sparse_loss)
    return jnp.stack([total, mask_loss, geometry_loss,
                      reprojection_loss, sparse_loss])
